# baseline (speedup 1.0000x reference)
import functools

import jax
import jax.numpy as jnp
from jax import lax
from jax.experimental import pallas as pl
from jax.experimental.pallas import tpu as pltpu
from jax.experimental.pallas import tpu_sc as plsc

NE = 8
TM = 256
TR = 512


def _router1_kernel(x_ref, gate_ref,
                    xa_ref, xb_ref, e0_ref, e1_ref, rnk0_ref, rnk1_ref,
                    cnt0_ref, cnt1_ref, carry0, carry1):
    g = pl.program_id(0)

    @pl.when(g == 0)
    def _init():
        carry0[...] = jnp.zeros_like(carry0)
        carry1[...] = jnp.zeros_like(carry1)

    x = x_ref[...]
    logits = jnp.dot(x, gate_ref[...], preferred_element_type=jnp.float32)
    ii = lax.broadcasted_iota(jnp.int32, logits.shape, 1)
    m1 = jnp.max(logits, axis=1, keepdims=True)
    e0 = jnp.min(jnp.where(logits == m1, ii, NE), axis=1, keepdims=True)
    l2 = jnp.where(ii == e0, -jnp.inf, logits)
    m2 = jnp.max(l2, axis=1, keepdims=True)
    e1 = jnp.min(jnp.where(l2 == m2, ii, NE), axis=1, keepdims=True)
    z = jnp.exp(m2 - m1)
    wb = z / (1.0 + z)
    xa_ref[...] = (1.0 - wb) * x
    xb_ref[...] = wb * x
    e0_ref[...] = e0
    e1_ref[...] = e1

    oh0 = (ii == e0).astype(jnp.float32)
    oh1 = (ii == e1).astype(jnp.float32)
    r_i = lax.broadcasted_iota(jnp.int32, (TR, TR), 0)
    r_j = lax.broadcasted_iota(jnp.int32, (TR, TR), 1)
    tril = (r_j < r_i).astype(jnp.float32)
    cum0 = jnp.dot(tril, oh0, preferred_element_type=jnp.float32)
    cum1 = jnp.dot(tril, oh1, preferred_element_type=jnp.float32)
    rnk0_ref[...] = jnp.sum((carry0[...] + cum0) * oh0, axis=1, keepdims=True)
    rnk1_ref[...] = jnp.sum((carry1[...] + cum1) * oh1, axis=1, keepdims=True)
    carry0[...] = carry0[...] + jnp.sum(oh0, axis=0, keepdims=True)
    carry1[...] = carry1[...] + jnp.sum(oh1, axis=0, keepdims=True)
    cnt0_ref[...] = carry0[...]
    cnt1_ref[...] = carry1[...]


def _router2_kernel(cnt0_ref, cnt1_ref, e0_ref, e1_ref, rnk0_ref, rnk1_ref,
                    pos0_ref, pos1_ref, eot_ref, base0, base1, *, num_tiles):
    g = pl.program_id(0)

    @pl.when(g == 0)
    def _phase2():
        counts = cnt0_ref[...] + cnt1_ref[...]
        padded = jnp.floor((counts + (TM - 1)) / TM) * TM
        pb = jnp.broadcast_to(padded, (NE, NE))
        cb = jnp.broadcast_to(cnt0_ref[...], (NE, NE))
        i_e = lax.broadcasted_iota(jnp.int32, (NE, NE), 0)
        i_k = lax.broadcasted_iota(jnp.int32, (NE, NE), 1)
        excl = jnp.sum(jnp.where(i_k < i_e, pb, 0.0), axis=1, keepdims=True)
        cnt0 = jnp.sum(jnp.where(i_k == i_e, cb, 0.0), axis=1, keepdims=True)
        base0[...] = excl
        base1[...] = excl + cnt0

    ii = lax.broadcasted_iota(jnp.int32, (TR, NE), 1)
    oh0 = (ii == e0_ref[...]).astype(jnp.float32)
    oh1 = (ii == e1_ref[...]).astype(jnp.float32)
    p0 = jnp.dot(oh0, base0[...], preferred_element_type=jnp.float32,
                 precision=lax.Precision.HIGHEST) + rnk0_ref[...]
    p1 = jnp.dot(oh1, base1[...], preferred_element_type=jnp.float32,
                 precision=lax.Precision.HIGHEST) + rnk1_ref[...]
    pos0_ref[...] = p0.astype(jnp.int32)
    pos1_ref[...] = p1.astype(jnp.int32)

    jt = lax.broadcasted_iota(jnp.int32, (NE, num_tiles), 1).astype(jnp.float32) * TM
    owned = (jt >= jnp.broadcast_to(base0[...], (NE, num_tiles))).astype(jnp.int32)
    eot_ref[...] = jnp.sum(owned, axis=0, keepdims=True) - 1


def _gmm_kernel(eot_ref, xs_ref, w1_ref, w2_ref, ys_ref):
    xb16 = xs_ref[...].astype(jnp.bfloat16)
    h = jnp.maximum(
        jnp.dot(xb16, w1_ref[0], preferred_element_type=jnp.float32), 0.0)
    ys_ref[...] = jnp.dot(h.astype(jnp.bfloat16), w2_ref[0],
                          preferred_element_type=jnp.float32)


def _dispatch_body(xa_hbm, xb_hbm, pos0_hbm, pos1_hbm, xs_hbm,
                   a0, a1, a2, b0, b1, b2, i00, i01, i02, i10, i11, i12,
                   idx0, idx1,
                   la0, la1, la2, lb0, lb1, lb2, sa0, sa1, sa2, sb0, sb1, sb2,
                   *, tok_per_w, chunk, d):
    nc = plsc.get_sparse_core_info().num_cores
    wid = lax.axis_index("s") * nc + lax.axis_index("c")
    nch = tok_per_w // chunk
    rowbase = wid * nch
    abufs, bbufs = (a0, a1, a2), (b0, b1, b2)
    i0bufs, i1bufs = (i00, i01, i02), (i10, i11, i12)
    lsems = ((la0, lb0), (la1, lb1), (la2, lb2))
    ssems = ((sa0, sb0), (sa1, sb1), (sa2, sb2))
    pltpu.sync_copy(pos0_hbm.at[pl.ds(rowbase, nch), :], idx0)
    pltpu.sync_copy(pos1_hbm.at[pl.ds(rowbase, nch), :], idx1)

    def load(c):
        s = c % 3
        base = wid * tok_per_w + c * chunk
        cpa = pltpu.async_copy(xa_hbm.at[pl.ds(base, chunk)], abufs[s], lsems[s][0])
        cpb = pltpu.async_copy(xb_hbm.at[pl.ds(base, chunk)], bbufs[s], lsems[s][1])
        return cpa, cpb

    pend_load = {0: load(0)}
    pend_scat = {}
    for c in range(nch):
        s = c % 3
        if c + 1 < nch:
            if c - 2 in pend_scat:
                wa_, wb_ = pend_scat.pop(c - 2)
                wa_.wait()
                wb_.wait()
            pend_load[c + 1] = load(c + 1)
        cpa, cpb = pend_load.pop(c)
        cpa.wait()
        cpb.wait()
        i0bufs[s][...] = idx0[c, :]
        i1bufs[s][...] = idx1[c, :]
        sc_a = pltpu.async_copy(abufs[s], xs_hbm.at[i0bufs[s]], ssems[s][0])
        sc_b = pltpu.async_copy(bbufs[s], xs_hbm.at[i1bufs[s]], ssems[s][1])
        pend_scat[c] = (sc_a, sc_b)
    for c in sorted(pend_scat):
        wa_, wb_ = pend_scat.pop(c)
        wa_.wait()
        wb_.wait()


def _combine_body(ys_hbm, pos0_hbm, pos1_hbm, out_hbm,
                  a0, a1, b0, b1, o0, o1, idx0, idx1,
                  ga0, ga1, gb0, gb1, so0, so1,
                  *, tok_per_w, chunk, d):
    nc = plsc.get_sparse_core_info().num_cores
    wid = lax.axis_index("s") * nc + lax.axis_index("c")
    lanes = 16
    nch = tok_per_w // chunk
    rowbase = wid * nch
    abufs, bbufs, obufs = (a0, a1), (b0, b1), (o0, o1)
    gsems = ((ga0, gb0), (ga1, gb1))
    osems = (so0, so1)
    pltpu.sync_copy(pos0_hbm.at[pl.ds(rowbase, nch), :], idx0)
    pltpu.sync_copy(pos1_hbm.at[pl.ds(rowbase, nch), :], idx1)

    def gather(c, s):
        cpa = pltpu.async_copy(ys_hbm.at[idx0.at[c]], abufs[s], gsems[s][0])
        cpb = pltpu.async_copy(ys_hbm.at[idx1.at[c]], bbufs[s], gsems[s][1])
        return cpa, cpb

    pend_g = {0: gather(0, 0)}
    pend_s = {}
    for c in range(nch):
        s = c % 2
        if c + 1 < nch:
            pend_g[c + 1] = gather(c + 1, (c + 1) % 2)
        cpa, cpb = pend_g.pop(c)
        cpa.wait()
        cpb.wait()
        if c - 2 in pend_s:
            pend_s.pop(c - 2).wait()
        abuf, bbuf, obuf = abufs[s], bbufs[s], obufs[s]

        def row_body(i, _):
            def lane_body(j, _):
                col = pl.ds(j * lanes, lanes)
                obuf[i, col] = abuf[i, col] + bbuf[i, col]
                return 0

            lax.fori_loop(0, d // lanes, lane_body, 0, unroll=16)
            return 0

        lax.fori_loop(0, chunk, row_body, 0)
        base = wid * tok_per_w + c * chunk
        pend_s[c] = pltpu.async_copy(obuf, out_hbm.at[pl.ds(base, chunk)], osems[s])
    for c in sorted(pend_s):
        pend_s.pop(c).wait()


def kernel(hidden_states, gate_w, w1, w2):
    b, s, d = hidden_states.shape
    ne, _, dff = w1.shape
    t = b * s
    x = hidden_states.reshape(t, d)

    nt_router = t // TR
    num_tiles = t * 2 // TM + NE
    p = num_tiles * TM

    r1 = pl.pallas_call(
        _router1_kernel,
        grid=(nt_router,),
        in_specs=[
            pl.BlockSpec((TR, d), lambda g: (g, 0)),
            pl.BlockSpec((d, NE), lambda g: (0, 0)),
        ],
        out_specs=[
            pl.BlockSpec((TR, d), lambda g: (g, 0)),
            pl.BlockSpec((TR, d), lambda g: (g, 0)),
            pl.BlockSpec((TR, 1), lambda g: (g, 0)),
            pl.BlockSpec((TR, 1), lambda g: (g, 0)),
            pl.BlockSpec((TR, 1), lambda g: (g, 0)),
            pl.BlockSpec((TR, 1), lambda g: (g, 0)),
            pl.BlockSpec((1, NE), lambda g: (0, 0)),
            pl.BlockSpec((1, NE), lambda g: (0, 0)),
        ],
        out_shape=[
            jax.ShapeDtypeStruct((t, d), jnp.float32),
            jax.ShapeDtypeStruct((t, d), jnp.float32),
            jax.ShapeDtypeStruct((t, 1), jnp.int32),
            jax.ShapeDtypeStruct((t, 1), jnp.int32),
            jax.ShapeDtypeStruct((t, 1), jnp.float32),
            jax.ShapeDtypeStruct((t, 1), jnp.float32),
            jax.ShapeDtypeStruct((1, NE), jnp.float32),
            jax.ShapeDtypeStruct((1, NE), jnp.float32),
        ],
        scratch_shapes=[
            pltpu.VMEM((1, NE), jnp.float32),
            pltpu.VMEM((1, NE), jnp.float32),
        ],
        compiler_params=pltpu.CompilerParams(
            dimension_semantics=("arbitrary",),
        ),
    )(x, gate_w)
    xa, xb, e0a, e1a, rnk0a, rnk1a, cnt0a, cnt1a = r1

    r2 = pl.pallas_call(
        functools.partial(_router2_kernel, num_tiles=num_tiles),
        grid=(nt_router,),
        in_specs=[
            pl.BlockSpec((1, NE), lambda g: (0, 0)),
            pl.BlockSpec((1, NE), lambda g: (0, 0)),
            pl.BlockSpec((TR, 1), lambda g: (g, 0)),
            pl.BlockSpec((TR, 1), lambda g: (g, 0)),
            pl.BlockSpec((TR, 1), lambda g: (g, 0)),
            pl.BlockSpec((TR, 1), lambda g: (g, 0)),
        ],
        out_specs=[
            pl.BlockSpec((TR, 1), lambda g: (g, 0)),
            pl.BlockSpec((TR, 1), lambda g: (g, 0)),
            pl.BlockSpec((1, num_tiles), lambda g: (0, 0)),
        ],
        out_shape=[
            jax.ShapeDtypeStruct((t, 1), jnp.int32),
            jax.ShapeDtypeStruct((t, 1), jnp.int32),
            jax.ShapeDtypeStruct((1, num_tiles), jnp.int32),
        ],
        scratch_shapes=[
            pltpu.VMEM((NE, 1), jnp.float32),
            pltpu.VMEM((NE, 1), jnp.float32),
        ],
        compiler_params=pltpu.CompilerParams(
            dimension_semantics=("arbitrary",),
        ),
    )(cnt0a, cnt1a, e0a, e1a, rnk0a, rnk1a)
    p02, p12, eot2 = r2
    pos0 = p02.reshape(t)
    pos1 = p12.reshape(t)
    eot = eot2.reshape(num_tiles)

    info = plsc.get_sparse_core_info()
    nw = info.num_cores * info.num_subcores
    tok_per_w = t // nw
    DCH = 16
    CCH = 16
    pos0_2d = pos0.reshape(t // DCH, DCH)
    pos1_2d = pos1.reshape(t // DCH, DCH)
    mesh = plsc.VectorSubcoreMesh(core_axis_name="c", subcore_axis_name="s")

    dispatch = functools.partial(
        pl.kernel,
        mesh=mesh,
        out_type=jax.ShapeDtypeStruct((p, d), jnp.float32),
        scratch_types=(
            [pltpu.VMEM((DCH, d), jnp.float32)] * 6
            + [pltpu.VMEM((DCH,), jnp.int32)] * 6
            + [pltpu.VMEM((tok_per_w // DCH, DCH), jnp.int32)] * 2
            + [pltpu.SemaphoreType.DMA] * 12),
    )(functools.partial(_dispatch_body, tok_per_w=tok_per_w, chunk=DCH, d=d))
    xs = dispatch(xa, xb, pos0_2d, pos1_2d)

    gmm_spec = pltpu.PrefetchScalarGridSpec(
        num_scalar_prefetch=1,
        grid=(num_tiles,),
        in_specs=[
            pl.BlockSpec((TM, d), lambda g, eot_ref: (g, 0)),
            pl.BlockSpec((1, d, dff), lambda g, eot_ref: (eot_ref[g], 0, 0)),
            pl.BlockSpec((1, dff, d), lambda g, eot_ref: (eot_ref[g], 0, 0)),
        ],
        out_specs=pl.BlockSpec((TM, d), lambda g, eot_ref: (g, 0)),
    )
    ys = pl.pallas_call(
        _gmm_kernel,
        grid_spec=gmm_spec,
        out_shape=jax.ShapeDtypeStruct((p, d), jnp.float32),
        compiler_params=pltpu.CompilerParams(
            dimension_semantics=("arbitrary",),
        ),
    )(eot, xs, w1.astype(jnp.bfloat16), w2.astype(jnp.bfloat16))

    combine = functools.partial(
        pl.kernel,
        mesh=mesh,
        out_type=jax.ShapeDtypeStruct((t, d), jnp.float32),
        scratch_types=[
            pltpu.VMEM((CCH, d), jnp.float32),
            pltpu.VMEM((CCH, d), jnp.float32),
            pltpu.VMEM((CCH, d), jnp.float32),
            pltpu.VMEM((CCH, d), jnp.float32),
            pltpu.VMEM((CCH, d), jnp.float32),
            pltpu.VMEM((CCH, d), jnp.float32),
            pltpu.VMEM((tok_per_w // CCH, CCH), jnp.int32),
            pltpu.VMEM((tok_per_w // CCH, CCH), jnp.int32),
        ] + [pltpu.SemaphoreType.DMA] * 6,
    )(functools.partial(_combine_body, tok_per_w=tok_per_w, chunk=CCH, d=d))
    out = combine(ys, pos0_2d, pos1_2d)
    return out.reshape(b, s, d)


# speedup vs baseline: 1.1753x; 1.1753x over previous
import functools

import jax
import jax.numpy as jnp
from jax import lax
from jax.experimental import pallas as pl
from jax.experimental.pallas import tpu as pltpu
from jax.experimental.pallas import tpu_sc as plsc

NE = 8
TM = 256
TR = 512


def _router1_kernel(x_ref, gate_ref,
                    xa_ref, xb_ref, e0_ref, e1_ref, rnk0_ref, rnk1_ref,
                    cnt0_ref, cnt1_ref, carry0, carry1):
    g = pl.program_id(0)

    @pl.when(g == 0)
    def _init():
        carry0[...] = jnp.zeros_like(carry0)
        carry1[...] = jnp.zeros_like(carry1)

    x = x_ref[...]
    logits = jnp.dot(x, gate_ref[...], preferred_element_type=jnp.float32)
    ii = lax.broadcasted_iota(jnp.int32, logits.shape, 1)
    m1 = jnp.max(logits, axis=1, keepdims=True)
    e0 = jnp.min(jnp.where(logits == m1, ii, NE), axis=1, keepdims=True)
    l2 = jnp.where(ii == e0, -jnp.inf, logits)
    m2 = jnp.max(l2, axis=1, keepdims=True)
    e1 = jnp.min(jnp.where(l2 == m2, ii, NE), axis=1, keepdims=True)
    z = jnp.exp(m2 - m1)
    wb = z / (1.0 + z)
    xa_ref[...] = (1.0 - wb) * x
    xb_ref[...] = wb * x
    e0_ref[...] = e0
    e1_ref[...] = e1

    oh0 = (ii == e0).astype(jnp.float32)
    oh1 = (ii == e1).astype(jnp.float32)
    r_i = lax.broadcasted_iota(jnp.int32, (TR, TR), 0)
    r_j = lax.broadcasted_iota(jnp.int32, (TR, TR), 1)
    tril = (r_j < r_i).astype(jnp.float32)
    cum0 = jnp.dot(tril, oh0, preferred_element_type=jnp.float32)
    cum1 = jnp.dot(tril, oh1, preferred_element_type=jnp.float32)
    rnk0_ref[...] = jnp.sum((carry0[...] + cum0) * oh0, axis=1, keepdims=True)
    rnk1_ref[...] = jnp.sum((carry1[...] + cum1) * oh1, axis=1, keepdims=True)
    carry0[...] = carry0[...] + jnp.sum(oh0, axis=0, keepdims=True)
    carry1[...] = carry1[...] + jnp.sum(oh1, axis=0, keepdims=True)
    cnt0_ref[...] = carry0[...]
    cnt1_ref[...] = carry1[...]


def _router2_kernel(cnt0_ref, cnt1_ref, e0_ref, e1_ref, rnk0_ref, rnk1_ref,
                    pos0_ref, pos1_ref, eot_ref, base0, base1, *, num_tiles):
    g = pl.program_id(0)

    @pl.when(g == 0)
    def _phase2():
        counts = cnt0_ref[...] + cnt1_ref[...]
        padded = jnp.floor((counts + (TM - 1)) / TM) * TM
        pb = jnp.broadcast_to(padded, (NE, NE))
        cb = jnp.broadcast_to(cnt0_ref[...], (NE, NE))
        i_e = lax.broadcasted_iota(jnp.int32, (NE, NE), 0)
        i_k = lax.broadcasted_iota(jnp.int32, (NE, NE), 1)
        excl = jnp.sum(jnp.where(i_k < i_e, pb, 0.0), axis=1, keepdims=True)
        cnt0 = jnp.sum(jnp.where(i_k == i_e, cb, 0.0), axis=1, keepdims=True)
        base0[...] = excl
        base1[...] = excl + cnt0

    ii = lax.broadcasted_iota(jnp.int32, (TR, NE), 1)
    oh0 = (ii == e0_ref[...]).astype(jnp.float32)
    oh1 = (ii == e1_ref[...]).astype(jnp.float32)
    p0 = jnp.dot(oh0, base0[...], preferred_element_type=jnp.float32,
                 precision=lax.Precision.HIGHEST) + rnk0_ref[...]
    p1 = jnp.dot(oh1, base1[...], preferred_element_type=jnp.float32,
                 precision=lax.Precision.HIGHEST) + rnk1_ref[...]
    pos0_ref[...] = p0.astype(jnp.int32)
    pos1_ref[...] = p1.astype(jnp.int32)

    jt = lax.broadcasted_iota(jnp.int32, (NE, num_tiles), 1).astype(jnp.float32) * TM
    owned = (jt >= jnp.broadcast_to(base0[...], (NE, num_tiles))).astype(jnp.int32)
    counts = cnt0_ref[...] + cnt1_ref[...]
    padded = jnp.floor((counts + (TM - 1)) / TM) * TM
    total = jnp.sum(padded, axis=1, keepdims=True)
    jt1 = lax.broadcasted_iota(jnp.int32, (1, num_tiles), 1).astype(jnp.float32) * TM
    used = (jt1 < jnp.broadcast_to(total, (1, num_tiles))).astype(jnp.int32)
    eot_ref[...] = jnp.concatenate(
        [jnp.sum(owned, axis=0, keepdims=True) - 1, used], axis=0)


def _gmm_kernel(eot_ref, xs_ref, w1_ref, w2_ref, ys_ref):
    g = pl.program_id(0)

    @pl.when(eot_ref[1, g] == 1)
    def _():
        h = jnp.maximum(
            jnp.dot(xs_ref[...], w1_ref[0], preferred_element_type=jnp.float32), 0.0)
        ys_ref[...] = jnp.dot(h, w2_ref[0], preferred_element_type=jnp.float32)


def _dispatch_body(xa_hbm, xb_hbm, pos0_hbm, pos1_hbm, xs_hbm,
                   a0, a1, a2, b0, b1, b2, i00, i01, i02, i10, i11, i12,
                   idx0, idx1,
                   la0, la1, la2, lb0, lb1, lb2, sa0, sa1, sa2, sb0, sb1, sb2,
                   *, tok_per_w, chunk, d):
    nc = plsc.get_sparse_core_info().num_cores
    wid = lax.axis_index("s") * nc + lax.axis_index("c")
    nch = tok_per_w // chunk
    rowbase = wid * nch
    abufs, bbufs = (a0, a1, a2), (b0, b1, b2)
    i0bufs, i1bufs = (i00, i01, i02), (i10, i11, i12)
    lsems = ((la0, lb0), (la1, lb1), (la2, lb2))
    ssems = ((sa0, sb0), (sa1, sb1), (sa2, sb2))
    pltpu.sync_copy(pos0_hbm.at[pl.ds(rowbase, nch), :], idx0)
    pltpu.sync_copy(pos1_hbm.at[pl.ds(rowbase, nch), :], idx1)

    def load(c):
        s = c % 3
        base = wid * tok_per_w + c * chunk
        cpa = pltpu.async_copy(xa_hbm.at[pl.ds(base, chunk)], abufs[s], lsems[s][0])
        cpb = pltpu.async_copy(xb_hbm.at[pl.ds(base, chunk)], bbufs[s], lsems[s][1])
        return cpa, cpb

    pend_load = {0: load(0)}
    pend_scat = {}
    for c in range(nch):
        s = c % 3
        if c + 1 < nch:
            if c - 2 in pend_scat:
                wa_, wb_ = pend_scat.pop(c - 2)
                wa_.wait()
                wb_.wait()
            pend_load[c + 1] = load(c + 1)
        cpa, cpb = pend_load.pop(c)
        cpa.wait()
        cpb.wait()
        i0bufs[s][...] = idx0[c, :]
        i1bufs[s][...] = idx1[c, :]
        sc_a = pltpu.async_copy(abufs[s], xs_hbm.at[i0bufs[s]], ssems[s][0])
        sc_b = pltpu.async_copy(bbufs[s], xs_hbm.at[i1bufs[s]], ssems[s][1])
        pend_scat[c] = (sc_a, sc_b)
    for c in sorted(pend_scat):
        wa_, wb_ = pend_scat.pop(c)
        wa_.wait()
        wb_.wait()


def _combine_body(ys_hbm, pos0_hbm, pos1_hbm, out_hbm,
                  a0, a1, b0, b1, o0, o1, idx0, idx1,
                  ga0, ga1, gb0, gb1, so0, so1,
                  *, tok_per_w, chunk, d):
    nc = plsc.get_sparse_core_info().num_cores
    wid = lax.axis_index("s") * nc + lax.axis_index("c")
    lanes = 16
    nch = tok_per_w // chunk
    rowbase = wid * nch
    abufs, bbufs, obufs = (a0, a1), (b0, b1), (o0, o1)
    gsems = ((ga0, gb0), (ga1, gb1))
    osems = (so0, so1)
    pltpu.sync_copy(pos0_hbm.at[pl.ds(rowbase, nch), :], idx0)
    pltpu.sync_copy(pos1_hbm.at[pl.ds(rowbase, nch), :], idx1)

    def gather(c, s):
        cpa = pltpu.async_copy(ys_hbm.at[idx0.at[c]], abufs[s], gsems[s][0])
        cpb = pltpu.async_copy(ys_hbm.at[idx1.at[c]], bbufs[s], gsems[s][1])
        return cpa, cpb

    pend_g = {0: gather(0, 0)}
    pend_s = {}
    for c in range(nch):
        s = c % 2
        if c + 1 < nch:
            pend_g[c + 1] = gather(c + 1, (c + 1) % 2)
        cpa, cpb = pend_g.pop(c)
        cpa.wait()
        cpb.wait()
        if c - 2 in pend_s:
            pend_s.pop(c - 2).wait()
        abuf, bbuf, obuf = abufs[s], bbufs[s], obufs[s]

        def row_body(i, _):
            def lane_body(j, _):
                col = pl.ds(j * lanes, lanes)
                obuf[i, col] = abuf[i, col] + bbuf[i, col]
                return 0

            lax.fori_loop(0, d // lanes, lane_body, 0, unroll=16)
            return 0

        lax.fori_loop(0, chunk, row_body, 0)
        base = wid * tok_per_w + c * chunk
        pend_s[c] = pltpu.async_copy(obuf, out_hbm.at[pl.ds(base, chunk)], osems[s])
    for c in sorted(pend_s):
        pend_s.pop(c).wait()


def kernel(hidden_states, gate_w, w1, w2):
    b, s, d = hidden_states.shape
    ne, _, dff = w1.shape
    t = b * s
    x = hidden_states.reshape(t, d)

    nt_router = t // TR
    num_tiles = t * 2 // TM + NE
    p = num_tiles * TM

    r1 = pl.pallas_call(
        _router1_kernel,
        grid=(nt_router,),
        in_specs=[
            pl.BlockSpec((TR, d), lambda g: (g, 0)),
            pl.BlockSpec((d, NE), lambda g: (0, 0)),
        ],
        out_specs=[
            pl.BlockSpec((TR, d), lambda g: (g, 0)),
            pl.BlockSpec((TR, d), lambda g: (g, 0)),
            pl.BlockSpec((TR, 1), lambda g: (g, 0)),
            pl.BlockSpec((TR, 1), lambda g: (g, 0)),
            pl.BlockSpec((TR, 1), lambda g: (g, 0)),
            pl.BlockSpec((TR, 1), lambda g: (g, 0)),
            pl.BlockSpec((1, NE), lambda g: (0, 0)),
            pl.BlockSpec((1, NE), lambda g: (0, 0)),
        ],
        out_shape=[
            jax.ShapeDtypeStruct((t, d), jnp.float32),
            jax.ShapeDtypeStruct((t, d), jnp.float32),
            jax.ShapeDtypeStruct((t, 1), jnp.int32),
            jax.ShapeDtypeStruct((t, 1), jnp.int32),
            jax.ShapeDtypeStruct((t, 1), jnp.float32),
            jax.ShapeDtypeStruct((t, 1), jnp.float32),
            jax.ShapeDtypeStruct((1, NE), jnp.float32),
            jax.ShapeDtypeStruct((1, NE), jnp.float32),
        ],
        scratch_shapes=[
            pltpu.VMEM((1, NE), jnp.float32),
            pltpu.VMEM((1, NE), jnp.float32),
        ],
        compiler_params=pltpu.CompilerParams(
            dimension_semantics=("arbitrary",),
        ),
    )(x, gate_w)
    xa, xb, e0a, e1a, rnk0a, rnk1a, cnt0a, cnt1a = r1

    r2 = pl.pallas_call(
        functools.partial(_router2_kernel, num_tiles=num_tiles),
        grid=(nt_router,),
        in_specs=[
            pl.BlockSpec((1, NE), lambda g: (0, 0)),
            pl.BlockSpec((1, NE), lambda g: (0, 0)),
            pl.BlockSpec((TR, 1), lambda g: (g, 0)),
            pl.BlockSpec((TR, 1), lambda g: (g, 0)),
            pl.BlockSpec((TR, 1), lambda g: (g, 0)),
            pl.BlockSpec((TR, 1), lambda g: (g, 0)),
        ],
        out_specs=[
            pl.BlockSpec((TR, 1), lambda g: (g, 0)),
            pl.BlockSpec((TR, 1), lambda g: (g, 0)),
            pl.BlockSpec((2, num_tiles), lambda g: (0, 0)),
        ],
        out_shape=[
            jax.ShapeDtypeStruct((t, 1), jnp.int32),
            jax.ShapeDtypeStruct((t, 1), jnp.int32),
            jax.ShapeDtypeStruct((2, num_tiles), jnp.int32),
        ],
        scratch_shapes=[
            pltpu.VMEM((NE, 1), jnp.float32),
            pltpu.VMEM((NE, 1), jnp.float32),
        ],
        compiler_params=pltpu.CompilerParams(
            dimension_semantics=("arbitrary",),
        ),
    )(cnt0a, cnt1a, e0a, e1a, rnk0a, rnk1a)
    p02, p12, eot = r2
    pos0 = p02.reshape(t)
    pos1 = p12.reshape(t)

    info = plsc.get_sparse_core_info()
    nw = info.num_cores * info.num_subcores
    tok_per_w = t // nw
    DCH = 16
    CCH = 16
    pos0_2d = pos0.reshape(t // DCH, DCH)
    pos1_2d = pos1.reshape(t // DCH, DCH)
    mesh = plsc.VectorSubcoreMesh(core_axis_name="c", subcore_axis_name="s")

    dispatch = functools.partial(
        pl.kernel,
        mesh=mesh,
        out_type=jax.ShapeDtypeStruct((p, d), jnp.float32),
        scratch_types=(
            [pltpu.VMEM((DCH, d), jnp.float32)] * 6
            + [pltpu.VMEM((DCH,), jnp.int32)] * 6
            + [pltpu.VMEM((tok_per_w // DCH, DCH), jnp.int32)] * 2
            + [pltpu.SemaphoreType.DMA] * 12),
    )(functools.partial(_dispatch_body, tok_per_w=tok_per_w, chunk=DCH, d=d))
    xs = dispatch(xa, xb, pos0_2d, pos1_2d)

    gmm_spec = pltpu.PrefetchScalarGridSpec(
        num_scalar_prefetch=1,
        grid=(num_tiles,),
        in_specs=[
            pl.BlockSpec((TM, d), lambda g, eot_ref: (g, 0)),
            pl.BlockSpec((1, d, dff), lambda g, eot_ref: (eot_ref[0, g], 0, 0)),
            pl.BlockSpec((1, dff, d), lambda g, eot_ref: (eot_ref[0, g], 0, 0)),
        ],
        out_specs=pl.BlockSpec((TM, d), lambda g, eot_ref: (g, 0)),
    )
    ys = pl.pallas_call(
        _gmm_kernel,
        grid_spec=gmm_spec,
        out_shape=jax.ShapeDtypeStruct((p, d), jnp.float32),
        compiler_params=pltpu.CompilerParams(
            dimension_semantics=("arbitrary",),
        ),
    )(eot, xs, w1, w2)

    combine = functools.partial(
        pl.kernel,
        mesh=mesh,
        out_type=jax.ShapeDtypeStruct((t, d), jnp.float32),
        scratch_types=[
            pltpu.VMEM((CCH, d), jnp.float32),
            pltpu.VMEM((CCH, d), jnp.float32),
            pltpu.VMEM((CCH, d), jnp.float32),
            pltpu.VMEM((CCH, d), jnp.float32),
            pltpu.VMEM((CCH, d), jnp.float32),
            pltpu.VMEM((CCH, d), jnp.float32),
            pltpu.VMEM((tok_per_w // CCH, CCH), jnp.int32),
            pltpu.VMEM((tok_per_w // CCH, CCH), jnp.int32),
        ] + [pltpu.SemaphoreType.DMA] * 6,
    )(functools.partial(_combine_body, tok_per_w=tok_per_w, chunk=CCH, d=d))
    out = combine(ys, pos0_2d, pos1_2d)
    return out.reshape(b, s, d)
